# baseline (speedup 1.0000x reference)
import functools

import jax
import jax.numpy as jnp
from jax import lax
from jax.experimental import pallas as pl
from jax.experimental.pallas import tpu as pltpu
from jax.experimental.pallas import tpu_sc as plsc

N = 50000
E = 800000
H = 64
OUT = 16
L = 3
DE = 7

NPAD = 50176
EPAD = 819200
CH = 128
SCH = 128
J = SCH // CH
ET = EPAD // 16
NSC = ET // SCH
RPT = NPAD // 16
BN = 512
GN = NPAD // BN
BE = 2048
GE = EPAD // BE


def _edge_body(idx_h, a0_h, a1_h, b0_h, b1_h, c0_h, c1_h, z32_h,
               s0_h, s1_h,
               s_sh, av0, bv0, cv0, av1, bv1, cv1, i0, i1,
               ise0, ise1, gse0, gse1, sse):
    cid = lax.axis_index("c")
    sid = lax.axis_index("s")
    r0 = sid * RPT

    pltpu.sync_copy(z32_h.at[pl.ds(r0, RPT)], s_sh.at[pl.ds(r0, RPT)])

    plsc.subcore_barrier()

    rbase = sid * (ET // CH)
    ebase = sid * ET

    def issue_idx(t, ib, sem):
        @pl.when(t < NSC)
        def _():
            rb = 2 * (rbase + t)
            pltpu.async_copy(idx_h.at[pl.ds(rb, 2)], ib, sem)

    def wait_idx(ib, sem):
        pltpu.make_async_copy(idx_h.at[pl.ds(0, 2)], ib, sem).wait()

    def issue_gathers(t, ib, a_v, b_v, c_v, sem, a_h, b_h, c_h):
        @pl.when(t < NSC)
        def _():
            base = ebase + t * SCH
            pltpu.async_copy(a_h.at[ib.at[0]], a_v, sem)
            pltpu.async_copy(b_h.at[ib.at[1]], b_v, sem)
            pltpu.async_copy(c_h.at[pl.ds(base, SCH)], c_v, sem)

    def wait_gathers(ib, a_v, b_v, c_v, sem, a_h, b_h, c_h):
        pltpu.make_async_copy(a_h.at[ib.at[0]], a_v, sem).wait()
        pltpu.make_async_copy(b_h.at[ib.at[1]], b_v, sem).wait()
        pltpu.make_async_copy(c_h.at[pl.ds(0, SCH)], c_v, sem).wait()

    def compute(a_v, b_v, c_v):
        def row4(i, c2):
            for u in range(4):
                ii = i * 4 + u
                for h0 in (0, 16):
                    sl = pl.ds(h0, 16)
                    c_v[ii, sl] = jnp.maximum(a_v[ii, sl] + b_v[ii, sl] + c_v[ii, sl], 0.0)
            return c2
        lax.fori_loop(0, SCH // 4, row4, 0)

    def scatter_sync(ib, c_v):
        pltpu.async_copy(c_v, s_sh.at[ib.at[0]], sse, add=True).wait()

    def gath0(t):
        @pl.when(cid == 0)
        def _():
            issue_gathers(t, i0, av0, bv0, cv0, gse0, a0_h, b0_h, c0_h)

        @pl.when(cid == 1)
        def _():
            issue_gathers(t, i0, av0, bv0, cv0, gse0, a1_h, b1_h, c1_h)

    def gath1(t):
        @pl.when(cid == 0)
        def _():
            issue_gathers(t, i1, av1, bv1, cv1, gse1, a0_h, b0_h, c0_h)

        @pl.when(cid == 1)
        def _():
            issue_gathers(t, i1, av1, bv1, cv1, gse1, a1_h, b1_h, c1_h)

    def wg0():
        @pl.when(cid == 0)
        def _():
            wait_gathers(i0, av0, bv0, cv0, gse0, a0_h, b0_h, c0_h)

        @pl.when(cid == 1)
        def _():
            wait_gathers(i0, av0, bv0, cv0, gse0, a1_h, b1_h, c1_h)

    def wg1():
        @pl.when(cid == 0)
        def _():
            wait_gathers(i1, av1, bv1, cv1, gse1, a0_h, b0_h, c0_h)

        @pl.when(cid == 1)
        def _():
            wait_gathers(i1, av1, bv1, cv1, gse1, a1_h, b1_h, c1_h)

    issue_idx(0, i0, ise0)
    issue_idx(1, i1, ise1)
    wait_idx(i0, ise0)
    gath0(0)

    def body(m, carry):
        t0 = 2 * m
        t1 = t0 + 1
        wait_idx(i1, ise1)
        gath1(t1)
        wg0()
        compute(av0, bv0, cv0)
        scatter_sync(i0, cv0)
        issue_idx(t0 + 2, i0, ise0)
        wg1()
        compute(av1, bv1, cv1)

        @pl.when(t0 + 2 < NSC)
        def _():
            wait_idx(i0, ise0)

        gath0(t0 + 2)
        scatter_sync(i1, cv1)
        issue_idx(t1 + 2, i1, ise1)
        return carry

    lax.fori_loop(0, NSC // 2, body, 0)

    plsc.subcore_barrier()

    @pl.when(cid == 0)
    def _():
        pltpu.sync_copy(s_sh.at[pl.ds(r0, RPT)], s0_h.at[pl.ds(r0, RPT)])

    @pl.when(cid == 1)
    def _():
        pltpu.sync_copy(s_sh.at[pl.ds(r0, RPT)], s1_h.at[pl.ds(r0, RPT)])


def _edge_pass(idx2, a0, a1, b0, b1, c0, c1, z32):
    mesh = plsc.VectorSubcoreMesh(core_axis_name="c", subcore_axis_name="s")
    f = pl.kernel(
        _edge_body,
        mesh=mesh,
        compiler_params=pltpu.CompilerParams(use_tc_tiling_on_sc=False),
        out_type=[
            jax.ShapeDtypeStruct((NPAD, 32), jnp.float32),
            jax.ShapeDtypeStruct((NPAD, 32), jnp.float32),
        ],
        scratch_types=[
            pltpu.VMEM_SHARED((NPAD, 32), jnp.float32),
            pltpu.VMEM((SCH, 32), jnp.float32),
            pltpu.VMEM((SCH, 32), jnp.float32),
            pltpu.VMEM((SCH, 32), jnp.float32),
            pltpu.VMEM((SCH, 32), jnp.float32),
            pltpu.VMEM((SCH, 32), jnp.float32),
            pltpu.VMEM((SCH, 32), jnp.float32),
            pltpu.VMEM((2, CH), jnp.int32),
            pltpu.VMEM((2, CH), jnp.int32),
            pltpu.SemaphoreType.DMA,
            pltpu.SemaphoreType.DMA,
            pltpu.SemaphoreType.DMA,
            pltpu.SemaphoreType.DMA,
            pltpu.SemaphoreType.DMA,
        ],
    )
    return f(idx2, a0, a1, b0, b1, c0, c1, z32)


DBLK = 25
DNB = 8


def _deg_body(dst_h, z1_h, o2_h, deg_h, dg_sh, idv, onesv, ise, sse):
    cid = lax.axis_index("c")
    sid = lax.axis_index("s")
    wid = cid * 16 + sid
    r0 = sid * RPT

    pltpu.sync_copy(z1_h.at[pl.ds(r0, RPT)], dg_sh.at[pl.ds(r0, RPT)])
    pltpu.sync_copy(o2_h, onesv)
    plsc.subcore_barrier()

    rbase = wid * (DBLK * DNB)

    def blk(b, carry):
        pltpu.sync_copy(dst_h.at[pl.ds(rbase + b * DBLK, DBLK)], idv)
        hs = [pltpu.async_copy(onesv, dg_sh.at[idv.at[j]], sse, add=True)
              for j in range(DBLK)]
        for h in hs:
            h.wait()
        return carry

    lax.fori_loop(0, DNB, blk, 0)

    plsc.subcore_barrier()

    @pl.when(cid == 0)
    def _():
        pltpu.sync_copy(dg_sh.at[pl.ds(r0, RPT)], deg_h.at[0].at[pl.ds(r0, RPT)])

    @pl.when(cid == 1)
    def _():
        pltpu.sync_copy(dg_sh.at[pl.ds(r0, RPT)], deg_h.at[1].at[pl.ds(r0, RPT)])


def _deg_pass(dst, z1, o2):
    mesh = plsc.VectorSubcoreMesh(core_axis_name="c", subcore_axis_name="s")
    f = pl.kernel(
        _deg_body,
        mesh=mesh,
        compiler_params=pltpu.CompilerParams(use_tc_tiling_on_sc=False),
        out_type=[
            jax.ShapeDtypeStruct((2, NPAD, 2), jnp.float32),
        ],
        scratch_types=[
            pltpu.VMEM_SHARED((NPAD, 2), jnp.float32),
            pltpu.VMEM((DBLK, CH), jnp.int32),
            pltpu.VMEM((CH, 2), jnp.float32),
            pltpu.SemaphoreType.DMA,
            pltpu.SemaphoreType.DMA,
        ],
    )
    return f(dst, z1, o2)


def _prep_body(x_r, ew_r, eb_r, wa0_r, wa1_r, wb0_r, wb1_r,
               h_r, a0_r, a1_r, b0_r, b1_r):
    h = jnp.dot(x_r[...], ew_r[...], preferred_element_type=jnp.float32) + eb_r[...]
    h_r[...] = h
    a0_r[...] = jnp.dot(h, wa0_r[...], preferred_element_type=jnp.float32)
    a1_r[...] = jnp.dot(h, wa1_r[...], preferred_element_type=jnp.float32)
    b0_r[...] = jnp.dot(h, wb0_r[...], preferred_element_type=jnp.float32)
    b1_r[...] = jnp.dot(h, wb1_r[...], preferred_element_type=jnp.float32)


def _edgec_body(ea_r, w_r, b_r, *c_refs):
    cf = jnp.dot(ea_r[...], w_r[...], preferred_element_type=jnp.float32) + b_r[...]
    for k in range(2):
        c_refs[k][...] = cf[:, k * 128:(k + 1) * 128]


def _upd_body(h_r, s0_r, s1_r, dg_r, w2a_r, w2b_r, b2_r,
              wa0_r, wa1_r, wb0_r, wb1_r,
              hn_r, a0_r, a1_r, b0_r, b1_r):
    agg = (jnp.dot(s0_r[...], w2a_r[...], preferred_element_type=jnp.float32)
           + jnp.dot(s1_r[...], w2b_r[...], preferred_element_type=jnp.float32)
           + jnp.dot(dg_r[...], b2_r[...], preferred_element_type=jnp.float32))
    hn = jnp.maximum(h_r[...] + agg, 0.0)
    hn_r[...] = hn
    a0_r[...] = jnp.dot(hn, wa0_r[...], preferred_element_type=jnp.float32)
    a1_r[...] = jnp.dot(hn, wa1_r[...], preferred_element_type=jnp.float32)
    b0_r[...] = jnp.dot(hn, wb0_r[...], preferred_element_type=jnp.float32)
    b1_r[...] = jnp.dot(hn, wb1_r[...], preferred_element_type=jnp.float32)


def _fin_body(h_r, s0_r, s1_r, dg_r, w2a_r, w2b_r, b2_r,
              ow1_r, ob1_r, ow2_r, ob2_r, o_r):
    agg = (jnp.dot(s0_r[...], w2a_r[...], preferred_element_type=jnp.float32)
           + jnp.dot(s1_r[...], w2b_r[...], preferred_element_type=jnp.float32)
           + jnp.dot(dg_r[...], b2_r[...], preferred_element_type=jnp.float32))
    hn = jnp.maximum(h_r[...] + agg, 0.0)
    t = jnp.maximum(jnp.dot(hn, ow1_r[...], preferred_element_type=jnp.float32)
                    + ob1_r[...], 0.0)
    o_r[...] = jnp.dot(t, ow2_r[...], preferred_element_type=jnp.float32) + ob2_r[...]


def _full(shape):
    return pl.BlockSpec(shape, lambda i: tuple(0 for _ in shape))


def _rows(bs, w):
    return pl.BlockSpec((bs, w), lambda i: (i, 0))


BQ = BN // 4
GQ = NPAD // BN


def _prep(x4, ew, eb, wa0, wa1, wb0, wb1):
    return pl.pallas_call(
        _prep_body,
        grid=(GQ,),
        in_specs=[_rows(BQ, 4), _full((4, 4 * H)), _full((1, 4 * H))]
        + [_full((4 * H, 128))] * 4,
        out_specs=[_rows(BQ, 4 * H)] + [_rows(BQ, 128)] * 4,
        out_shape=[jax.ShapeDtypeStruct((NPAD // 4, 4 * H), jnp.float32)]
        + [jax.ShapeDtypeStruct((NPAD // 4, 128), jnp.float32)] * 4,
    )(x4, ew, eb, wa0, wa1, wb0, wb1)


def _edgec(ea4, w, b):
    nb = (E // 4 + BE // 4 - 1) // (BE // 4) - 1
    return pl.pallas_call(
        _edgec_body,
        grid=(GE,),
        in_specs=[pl.BlockSpec((BE // 4, 4 * DE), lambda i: (jnp.minimum(i, nb), 0)),
                  _full((4 * DE, 2 * 128)), _full((1, 2 * 128))],
        out_specs=[_rows(BE // 4, 128)] * 2,
        out_shape=[jax.ShapeDtypeStruct((EPAD // 4, 128), jnp.float32)] * 2,
    )(ea4, w, b)


def _upd(h, s0, s1, dg, w2a, w2b, b2, wa0, wa1, wb0, wb1):
    return pl.pallas_call(
        _upd_body,
        grid=(GQ,),
        in_specs=[_rows(BQ, 4 * H), _rows(BQ, 128), _rows(BQ, 128), _rows(BQ, 4),
                  _full((128, 4 * H)), _full((128, 4 * H)), _full((4, 4 * H))]
        + [_full((4 * H, 128))] * 4,
        out_specs=[_rows(BQ, 4 * H)] + [_rows(BQ, 128)] * 4,
        out_shape=[jax.ShapeDtypeStruct((NPAD // 4, 4 * H), jnp.float32)]
        + [jax.ShapeDtypeStruct((NPAD // 4, 128), jnp.float32)] * 4,
    )(h, s0, s1, dg, w2a, w2b, b2, wa0, wa1, wb0, wb1)


def _fin(h, s0, s1, dg, w2a, w2b, b2, ow1, ob1, ow2, ob2):
    return pl.pallas_call(
        _fin_body,
        grid=(GQ,),
        in_specs=[_rows(BQ, 4 * H), _rows(BQ, 128), _rows(BQ, 128), _rows(BQ, 4),
                  _full((128, 4 * H)), _full((128, 4 * H)), _full((4, 4 * H)),
                  _full((4 * H, 4 * OUT)), _full((1, 4 * OUT)),
                  _full((4 * OUT, 4)), _full((1, 4))],
        out_specs=[_rows(BQ, 4)],
        out_shape=[jax.ShapeDtypeStruct((NPAD // 4, 4), jnp.float32)],
    )(h, s0, s1, dg, w2a, w2b, b2, ow1, ob1, ow2, ob2)


def _kron4(w):
    return jnp.kron(jnp.eye(4, dtype=jnp.float32), w)


def _tile4(b):
    return jnp.tile(b.reshape(1, -1), (1, 4))


def kernel(x, edge_index, edge_attr, embed_W, embed_b, W1, b1, W2, b2,
           out_W1, out_b1, out_W2, out_b2):
    src = edge_index[0]
    dst = edge_index[1]
    dstp = jnp.concatenate([dst, jnp.full((EPAD - E,), N, jnp.int32)]).reshape(EPAD // CH, CH)
    srcp = jnp.concatenate([src, jnp.zeros((EPAD - E,), jnp.int32)]).reshape(EPAD // CH, CH)
    idx2 = jnp.stack([dstp, srcp], axis=1).reshape(2 * (EPAD // CH), CH)
    xp = jnp.concatenate([x, jnp.zeros((NPAD - N, 1), jnp.float32)])
    x4 = xp.reshape(NPAD // 4, 4)
    ea4 = edge_attr.reshape(E // 4, 4 * DE)

    ew4 = _kron4(embed_W)
    eb4 = _tile4(embed_b)
    wa0 = [_kron4(W1[l, :H, 0:32]) for l in range(L)]
    wa1 = [_kron4(W1[l, :H, 32:64]) for l in range(L)]
    wb0 = [_kron4(W1[l, H:2 * H, 0:32]) for l in range(L)]
    wb1 = [_kron4(W1[l, H:2 * H, 32:64]) for l in range(L)]
    w1e4 = [jnp.concatenate(
        [_kron4(W1[l, 2 * H:, c * 32:(c + 1) * 32]) for c in range(2)], axis=1)
        for l in range(L)]
    b1e4 = [jnp.concatenate(
        [_tile4(b1[l, c * 32:(c + 1) * 32]) for c in range(2)], axis=1)
        for l in range(L)]
    w2a = [_kron4(W2[l, :32, :]) for l in range(L)]
    w2b = [_kron4(W2[l, 32:, :]) for l in range(L)]
    b24 = [_kron4(b2[l].reshape(1, H)) for l in range(L)]
    ow14 = _kron4(out_W1)
    ob14 = _tile4(out_b1)
    ow24 = _kron4(out_W2)
    ob24 = _tile4(out_b2)
    z32 = jnp.zeros((NPAD, 32), jnp.float32)
    z1 = jnp.zeros((NPAD, 2), jnp.float32)
    o2 = jnp.ones((CH, 2), jnp.float32)

    cs = []
    for l in range(L):
        c0, c1 = _edgec(ea4, w1e4[l], b1e4[l])
        cs += [c0.reshape(EPAD, 32), c1.reshape(EPAD, 32)]

    (deg,) = _deg_pass(dstp, z1, o2)
    dg = (deg[0, :, 0] + deg[1, :, 0]).reshape(NPAD // 4, 4)

    h, a0, a1, bb0, bb1 = _prep(x4, ew4, eb4, wa0[0], wa1[0], wb0[0], wb1[0])
    for l in range(L):
        s0, s1 = _edge_pass(idx2,
                            a0.reshape(NPAD, 32), a1.reshape(NPAD, 32),
                            bb0.reshape(NPAD, 32), bb1.reshape(NPAD, 32),
                            cs[2 * l], cs[2 * l + 1], z32)
        s0t = s0.reshape(NPAD // 4, 128)
        s1t = s1.reshape(NPAD // 4, 128)
        if l < L - 1:
            h, a0, a1, bb0, bb1 = _upd(h, s0t, s1t, dg, w2a[l], w2b[l], b24[l],
                                       wa0[l + 1], wa1[l + 1],
                                       wb0[l + 1], wb1[l + 1])
        else:
            (out,) = _fin(h, s0t, s1t, dg, w2a[l], w2b[l], b24[l],
                          ow14, ob14, ow24, ob24)
    return out.reshape(NPAD, 1)[:N]


# speedup vs baseline: 1.1487x; 1.1487x over previous
import functools

import jax
import jax.numpy as jnp
from jax import lax
from jax.experimental import pallas as pl
from jax.experimental.pallas import tpu as pltpu
from jax.experimental.pallas import tpu_sc as plsc

N = 50000
E = 800000
H = 64
OUT = 16
L = 3
DE = 7

NPAD = 50176
EPAD = 819200
CH = 128
SCH = 128
J = SCH // CH
ET = EPAD // 16
NSC = ET // SCH
RPT = NPAD // 16
BN = 512
GN = NPAD // BN
BE = 2048
GE = EPAD // BE


def _edge_body(idx_h, a0_h, a1_h, b0_h, b1_h, c0_h, c1_h, z32_h,
               s0_h, s1_h,
               s_sh, av0, bv0, cv0, av1, bv1, cv1, i0, i1,
               ise0, ise1, gse0, gse1, sse):
    cid = lax.axis_index("c")
    sid = lax.axis_index("s")
    r0 = sid * RPT

    pltpu.sync_copy(z32_h.at[pl.ds(r0, RPT)], s_sh.at[pl.ds(r0, RPT)])

    plsc.subcore_barrier()

    rbase = sid * (ET // CH)
    ebase = sid * ET

    def issue_pair(t, pb, sem):
        @pl.when(t < NSC)
        def _():
            rb = 2 * (rbase + t)
            pltpu.async_copy(idx_h.at[pl.ds(rb, 4)], pb, sem)

    def wait_pair(pb, sem):
        pltpu.make_async_copy(idx_h.at[pl.ds(0, 4)], pb, sem).wait()

    def _issue_g(t, pb, o, a_v, b_v, c_v, sem, a_h, b_h, c_h):
        @pl.when(t < NSC)
        def _():
            base = ebase + t * SCH
            pltpu.async_copy(a_h.at[pb.at[o]], a_v, sem)
            pltpu.async_copy(b_h.at[pb.at[o + 1]], b_v, sem)
            pltpu.async_copy(c_h.at[pl.ds(base, SCH)], c_v, sem)

    def _wait_g(pb, o, a_v, b_v, c_v, sem, a_h, b_h, c_h):
        pltpu.make_async_copy(a_h.at[pb.at[o]], a_v, sem).wait()
        pltpu.make_async_copy(b_h.at[pb.at[o + 1]], b_v, sem).wait()
        pltpu.make_async_copy(c_h.at[pl.ds(0, SCH)], c_v, sem).wait()

    def compute(a_v, b_v, c_v):
        def row4(i, c2):
            for u in range(4):
                ii = i * 4 + u
                for h0 in (0, 16):
                    sl = pl.ds(h0, 16)
                    c_v[ii, sl] = jnp.maximum(a_v[ii, sl] + b_v[ii, sl] + c_v[ii, sl], 0.0)
            return c2
        lax.fori_loop(0, SCH // 4, row4, 0)

    DAT = ((av0, bv0, cv0, gse0), (av1, bv1, cv1, gse1))

    def gath(t, pb, o, slot):
        a_v, b_v, c_v, sem = DAT[slot]

        @pl.when(cid == 0)
        def _():
            _issue_g(t, pb, o, a_v, b_v, c_v, sem, a0_h, b0_h, c0_h)

        @pl.when(cid == 1)
        def _():
            _issue_g(t, pb, o, a_v, b_v, c_v, sem, a1_h, b1_h, c1_h)

    def step(t, pb, o, slot):
        a_v, b_v, c_v, sem = DAT[slot]

        @pl.when(cid == 0)
        def _():
            _wait_g(pb, o, a_v, b_v, c_v, sem, a0_h, b0_h, c0_h)

        @pl.when(cid == 1)
        def _():
            _wait_g(pb, o, a_v, b_v, c_v, sem, a1_h, b1_h, c1_h)

        compute(a_v, b_v, c_v)
        pltpu.async_copy(c_v, s_sh.at[pb.at[o]], sse, add=True).wait()

    issue_pair(0, i0, ise0)
    issue_pair(2, i1, ise1)
    wait_pair(i0, ise0)
    gath(0, i0, 0, 0)

    def body(mm, carry):
        t = 4 * mm
        gath(t + 1, i0, 2, 1)
        step(t, i0, 0, 0)
        wait_pair(i1, ise1)
        gath(t + 2, i1, 0, 0)
        step(t + 1, i0, 2, 1)
        issue_pair(t + 4, i0, ise0)
        gath(t + 3, i1, 2, 1)
        step(t + 2, i1, 0, 0)

        @pl.when(t + 4 < NSC)
        def _():
            wait_pair(i0, ise0)

        gath(t + 4, i0, 0, 0)
        step(t + 3, i1, 2, 1)
        issue_pair(t + 6, i1, ise1)
        return carry

    lax.fori_loop(0, NSC // 4, body, 0)

    plsc.subcore_barrier()

    @pl.when(cid == 0)
    def _():
        pltpu.sync_copy(s_sh.at[pl.ds(r0, RPT)], s0_h.at[pl.ds(r0, RPT)])

    @pl.when(cid == 1)
    def _():
        pltpu.sync_copy(s_sh.at[pl.ds(r0, RPT)], s1_h.at[pl.ds(r0, RPT)])


def _edge_pass(idx2, a0, a1, b0, b1, c0, c1, z32):
    mesh = plsc.VectorSubcoreMesh(core_axis_name="c", subcore_axis_name="s")
    f = pl.kernel(
        _edge_body,
        mesh=mesh,
        compiler_params=pltpu.CompilerParams(use_tc_tiling_on_sc=False),
        out_type=[
            jax.ShapeDtypeStruct((NPAD, 32), jnp.float32),
            jax.ShapeDtypeStruct((NPAD, 32), jnp.float32),
        ],
        scratch_types=[
            pltpu.VMEM_SHARED((NPAD, 32), jnp.float32),
            pltpu.VMEM((SCH, 32), jnp.float32),
            pltpu.VMEM((SCH, 32), jnp.float32),
            pltpu.VMEM((SCH, 32), jnp.float32),
            pltpu.VMEM((SCH, 32), jnp.float32),
            pltpu.VMEM((SCH, 32), jnp.float32),
            pltpu.VMEM((SCH, 32), jnp.float32),
            pltpu.VMEM((4, CH), jnp.int32),
            pltpu.VMEM((4, CH), jnp.int32),
            pltpu.SemaphoreType.DMA,
            pltpu.SemaphoreType.DMA,
            pltpu.SemaphoreType.DMA,
            pltpu.SemaphoreType.DMA,
            pltpu.SemaphoreType.DMA,
        ],
    )
    return f(idx2, a0, a1, b0, b1, c0, c1, z32)


DBLK = 25
DNB = 8


def _deg_body(dst_h, z1_h, o2_h, deg_h, dg_sh, idv, onesv, ise, sse):
    cid = lax.axis_index("c")
    sid = lax.axis_index("s")
    wid = cid * 16 + sid
    r0 = sid * RPT

    pltpu.sync_copy(z1_h.at[pl.ds(r0, RPT)], dg_sh.at[pl.ds(r0, RPT)])
    pltpu.sync_copy(o2_h, onesv)
    plsc.subcore_barrier()

    rbase = wid * (DBLK * DNB)

    def blk(b, carry):
        pltpu.sync_copy(dst_h.at[pl.ds(rbase + b * DBLK, DBLK)], idv)
        hs = [pltpu.async_copy(onesv, dg_sh.at[idv.at[j]], sse, add=True)
              for j in range(DBLK)]
        for h in hs:
            h.wait()
        return carry

    lax.fori_loop(0, DNB, blk, 0)

    plsc.subcore_barrier()

    @pl.when(cid == 0)
    def _():
        pltpu.sync_copy(dg_sh.at[pl.ds(r0, RPT)], deg_h.at[0].at[pl.ds(r0, RPT)])

    @pl.when(cid == 1)
    def _():
        pltpu.sync_copy(dg_sh.at[pl.ds(r0, RPT)], deg_h.at[1].at[pl.ds(r0, RPT)])


def _deg_pass(dst, z1, o2):
    mesh = plsc.VectorSubcoreMesh(core_axis_name="c", subcore_axis_name="s")
    f = pl.kernel(
        _deg_body,
        mesh=mesh,
        compiler_params=pltpu.CompilerParams(use_tc_tiling_on_sc=False),
        out_type=[
            jax.ShapeDtypeStruct((2, NPAD, 2), jnp.float32),
        ],
        scratch_types=[
            pltpu.VMEM_SHARED((NPAD, 2), jnp.float32),
            pltpu.VMEM((DBLK, CH), jnp.int32),
            pltpu.VMEM((CH, 2), jnp.float32),
            pltpu.SemaphoreType.DMA,
            pltpu.SemaphoreType.DMA,
        ],
    )
    return f(dst, z1, o2)


def _prep_body(x_r, ew_r, eb_r, wa0_r, wa1_r, wb0_r, wb1_r,
               h_r, a0_r, a1_r, b0_r, b1_r):
    h = jnp.dot(x_r[...], ew_r[...], preferred_element_type=jnp.float32) + eb_r[...]
    h_r[...] = h
    a0_r[...] = jnp.dot(h, wa0_r[...], preferred_element_type=jnp.float32)
    a1_r[...] = jnp.dot(h, wa1_r[...], preferred_element_type=jnp.float32)
    b0_r[...] = jnp.dot(h, wb0_r[...], preferred_element_type=jnp.float32)
    b1_r[...] = jnp.dot(h, wb1_r[...], preferred_element_type=jnp.float32)


def _edgec_body(ea_r, w_r, b_r, *c_refs):
    cf = jnp.dot(ea_r[...], w_r[...], preferred_element_type=jnp.float32) + b_r[...]
    for k in range(2 * L):
        c_refs[k][...] = cf[:, k * 128:(k + 1) * 128]


def _upd_body(h_r, s0_r, s1_r, dg_r, w2a_r, w2b_r, b2_r,
              wa0_r, wa1_r, wb0_r, wb1_r,
              hn_r, a0_r, a1_r, b0_r, b1_r):
    agg = (jnp.dot(s0_r[...], w2a_r[...], preferred_element_type=jnp.float32)
           + jnp.dot(s1_r[...], w2b_r[...], preferred_element_type=jnp.float32)
           + jnp.dot(dg_r[...], b2_r[...], preferred_element_type=jnp.float32))
    hn = jnp.maximum(h_r[...] + agg, 0.0)
    hn_r[...] = hn
    a0_r[...] = jnp.dot(hn, wa0_r[...], preferred_element_type=jnp.float32)
    a1_r[...] = jnp.dot(hn, wa1_r[...], preferred_element_type=jnp.float32)
    b0_r[...] = jnp.dot(hn, wb0_r[...], preferred_element_type=jnp.float32)
    b1_r[...] = jnp.dot(hn, wb1_r[...], preferred_element_type=jnp.float32)


def _fin_body(h_r, s0_r, s1_r, dg_r, w2a_r, w2b_r, b2_r,
              ow1_r, ob1_r, ow2_r, ob2_r, o_r):
    agg = (jnp.dot(s0_r[...], w2a_r[...], preferred_element_type=jnp.float32)
           + jnp.dot(s1_r[...], w2b_r[...], preferred_element_type=jnp.float32)
           + jnp.dot(dg_r[...], b2_r[...], preferred_element_type=jnp.float32))
    hn = jnp.maximum(h_r[...] + agg, 0.0)
    t = jnp.maximum(jnp.dot(hn, ow1_r[...], preferred_element_type=jnp.float32)
                    + ob1_r[...], 0.0)
    o_r[...] = jnp.dot(t, ow2_r[...], preferred_element_type=jnp.float32) + ob2_r[...]


def _full(shape):
    return pl.BlockSpec(shape, lambda i: tuple(0 for _ in shape))


def _rows(bs, w):
    return pl.BlockSpec((bs, w), lambda i: (i, 0))


BQ = BN // 4
GQ = NPAD // BN


def _prep(x4, ew, eb, wa0, wa1, wb0, wb1):
    return pl.pallas_call(
        _prep_body,
        grid=(GQ,),
        in_specs=[_rows(BQ, 4), _full((4, 4 * H)), _full((1, 4 * H))]
        + [_full((4 * H, 128))] * 4,
        out_specs=[_rows(BQ, 4 * H)] + [_rows(BQ, 128)] * 4,
        out_shape=[jax.ShapeDtypeStruct((NPAD // 4, 4 * H), jnp.float32)]
        + [jax.ShapeDtypeStruct((NPAD // 4, 128), jnp.float32)] * 4,
    )(x4, ew, eb, wa0, wa1, wb0, wb1)


def _edgec(ea4, w, b):
    nb = (E // 4 + BE // 4 - 1) // (BE // 4) - 1
    return pl.pallas_call(
        _edgec_body,
        grid=(GE,),
        in_specs=[pl.BlockSpec((BE // 4, 4 * DE), lambda i: (jnp.minimum(i, nb), 0)),
                  _full((4 * DE, 2 * L * 128)), _full((1, 2 * L * 128))],
        out_specs=[_rows(BE // 4, 128)] * (2 * L),
        out_shape=[jax.ShapeDtypeStruct((EPAD // 4, 128), jnp.float32)] * (2 * L),
    )(ea4, w, b)


def _upd(h, s0, s1, dg, w2a, w2b, b2, wa0, wa1, wb0, wb1):
    return pl.pallas_call(
        _upd_body,
        grid=(GQ,),
        in_specs=[_rows(BQ, 4 * H), _rows(BQ, 128), _rows(BQ, 128), _rows(BQ, 4),
                  _full((128, 4 * H)), _full((128, 4 * H)), _full((4, 4 * H))]
        + [_full((4 * H, 128))] * 4,
        out_specs=[_rows(BQ, 4 * H)] + [_rows(BQ, 128)] * 4,
        out_shape=[jax.ShapeDtypeStruct((NPAD // 4, 4 * H), jnp.float32)]
        + [jax.ShapeDtypeStruct((NPAD // 4, 128), jnp.float32)] * 4,
    )(h, s0, s1, dg, w2a, w2b, b2, wa0, wa1, wb0, wb1)


def _fin(h, s0, s1, dg, w2a, w2b, b2, ow1, ob1, ow2, ob2):
    return pl.pallas_call(
        _fin_body,
        grid=(GQ,),
        in_specs=[_rows(BQ, 4 * H), _rows(BQ, 128), _rows(BQ, 128), _rows(BQ, 4),
                  _full((128, 4 * H)), _full((128, 4 * H)), _full((4, 4 * H)),
                  _full((4 * H, 4 * OUT)), _full((1, 4 * OUT)),
                  _full((4 * OUT, 4)), _full((1, 4))],
        out_specs=[_rows(BQ, 4)],
        out_shape=[jax.ShapeDtypeStruct((NPAD // 4, 4), jnp.float32)],
    )(h, s0, s1, dg, w2a, w2b, b2, ow1, ob1, ow2, ob2)


def _kron4(w):
    return jnp.kron(jnp.eye(4, dtype=jnp.float32), w)


def _tile4(b):
    return jnp.tile(b.reshape(1, -1), (1, 4))


def kernel(x, edge_index, edge_attr, embed_W, embed_b, W1, b1, W2, b2,
           out_W1, out_b1, out_W2, out_b2):
    src = edge_index[0]
    dst = edge_index[1]
    dstp = jnp.concatenate([dst, jnp.full((EPAD - E,), N, jnp.int32)]).reshape(EPAD // CH, CH)
    srcp = jnp.concatenate([src, jnp.zeros((EPAD - E,), jnp.int32)]).reshape(EPAD // CH, CH)
    idx2 = jnp.stack([dstp, srcp], axis=1).reshape(2 * (EPAD // CH), CH)
    xp = jnp.concatenate([x, jnp.zeros((NPAD - N, 1), jnp.float32)])
    x4 = xp.reshape(NPAD // 4, 4)
    ea4 = edge_attr.reshape(E // 4, 4 * DE)

    ew4 = _kron4(embed_W)
    eb4 = _tile4(embed_b)
    wa0 = [_kron4(W1[l, :H, 0:32]) for l in range(L)]
    wa1 = [_kron4(W1[l, :H, 32:64]) for l in range(L)]
    wb0 = [_kron4(W1[l, H:2 * H, 0:32]) for l in range(L)]
    wb1 = [_kron4(W1[l, H:2 * H, 32:64]) for l in range(L)]
    w1e4 = jnp.concatenate(
        [_kron4(W1[l, 2 * H:, c * 32:(c + 1) * 32])
         for l in range(L) for c in range(2)], axis=1)
    b1e4 = jnp.concatenate(
        [_tile4(b1[l, c * 32:(c + 1) * 32])
         for l in range(L) for c in range(2)], axis=1)
    w2a = [_kron4(W2[l, :32, :]) for l in range(L)]
    w2b = [_kron4(W2[l, 32:, :]) for l in range(L)]
    b24 = [_kron4(b2[l].reshape(1, H)) for l in range(L)]
    ow14 = _kron4(out_W1)
    ob14 = _tile4(out_b1)
    ow24 = _kron4(out_W2)
    ob24 = _tile4(out_b2)
    z32 = jnp.zeros((NPAD, 32), jnp.float32)
    z1 = jnp.zeros((NPAD, 2), jnp.float32)
    o2 = jnp.ones((CH, 2), jnp.float32)

    cs = _edgec(ea4, w1e4, b1e4)
    cs = [c.reshape(EPAD, 32) for c in cs]

    (deg,) = _deg_pass(dstp, z1, o2)
    dg = (deg[0, :, 0] + deg[1, :, 0]).reshape(NPAD // 4, 4)

    h, a0, a1, bb0, bb1 = _prep(x4, ew4, eb4, wa0[0], wa1[0], wb0[0], wb1[0])
    for l in range(L):
        s0, s1 = _edge_pass(idx2,
                            a0.reshape(NPAD, 32), a1.reshape(NPAD, 32),
                            bb0.reshape(NPAD, 32), bb1.reshape(NPAD, 32),
                            cs[2 * l], cs[2 * l + 1], z32)
        s0t = s0.reshape(NPAD // 4, 128)
        s1t = s1.reshape(NPAD // 4, 128)
        if l < L - 1:
            h, a0, a1, bb0, bb1 = _upd(h, s0t, s1t, dg, w2a[l], w2b[l], b24[l],
                                       wa0[l + 1], wa1[l + 1],
                                       wb0[l + 1], wb1[l + 1])
        else:
            (out,) = _fin(h, s0t, s1t, dg, w2a[l], w2b[l], b24[l],
                          ow14, ob14, ow24, ob24)
    return out.reshape(NPAD, 1)[:N]
